# baseline (speedup 1.0000x reference)
import functools

import numpy as np
import jax
import jax.numpy as jnp
from jax.experimental import pallas as pl
from jax.experimental.pallas import tpu as pltpu


def _bottleneck_body(x_ref, w1_ref, w2_ref, w3_ref, sb12_ref, sb3_ref,
                     o_ref, hpad_ref, r2_ref, *, d_size, h_size, w_size):
    b, cin, s_size = x_ref.shape
    p = w1_ref.shape[1]
    cdt = r2_ref.dtype
    f32 = jnp.float32
    dn_b = (((1,), (1,)), ((0,), (0,)))

    x = x_ref[...]

    w1b = jnp.broadcast_to(w1_ref[...][None], (b, cin, p))
    h1 = jax.lax.dot_general(w1b, x.astype(cdt), dn_b,
                             preferred_element_type=f32)
    h1 = jnp.maximum(h1 * sb12_ref[:, 0:1][None] + sb12_ref[:, 1:2][None], 0.0)

    hpad_ref[:, :, 0:128] = jnp.zeros((b, p, 128), cdt)
    hpad_ref[:, :, s_size + 128:s_size + 256] = jnp.zeros((b, p, 128), cdt)
    hpad_ref[:, :, 128:s_size + 128] = h1.astype(cdt)

    lane = jax.lax.broadcasted_iota(jnp.int32, (1, 1, s_size), 2)
    h_of_lane = (lane // w_size) % h_size
    w_of_lane = lane % w_size

    for kd in range(3):
        for kh in range(3):
            t = kd * 3 + kh
            off = 128 + (kd - 1) * h_size * w_size + (kh - 1) * w_size
            src = hpad_ref[:, :, off:off + s_size]
            if kh == 0:
                src = jnp.where(h_of_lane != 0, src, 0)
            elif kh == 2:
                src = jnp.where(h_of_lane != h_size - 1, src, 0)
            r2_ref[:, t * p:(t + 1) * p, :] = src

    r2 = r2_ref[...]
    kp = 9 * p
    w2b = jnp.broadcast_to(w2_ref[...][None], (b, kp, 3 * p))
    y_all = jax.lax.dot_general(w2b, r2, dn_b,
                                preferred_element_type=f32)
    y0 = y_all[:, 0:p]
    y1 = y_all[:, p:2 * p]
    y2 = y_all[:, 2 * p:3 * p]

    h2 = y1
    h2 = h2 + jnp.where(w_of_lane != 0, jnp.roll(y0, 1, axis=2), 0.0)
    h2 = h2 + jnp.where(w_of_lane != w_size - 1, jnp.roll(y2, -1, axis=2), 0.0)
    h2 = jnp.maximum(h2 * sb12_ref[:, 2:3][None] + sb12_ref[:, 3:4][None], 0.0)

    w3b = jnp.broadcast_to(w3_ref[...][None], (b, p, cin))
    h3 = jax.lax.dot_general(w3b, h2.astype(cdt), dn_b,
                             preferred_element_type=f32)
    h3 = h3 * sb3_ref[:, 0:1][None] + sb3_ref[:, 1:2][None]
    o_ref[...] = jnp.maximum(h3 + x, 0.0).astype(o_ref.dtype)


def kernel(x, w1p, s1p, b1p, w2f, s2t, b2t, w3b, s3t, b3t):
    N, Cin, D, H, W = x.shape
    S = D * H * W
    P = w2f.shape[1] // (H * W)
    Wp = W + 2
    rowp = w1p.shape[1]
    cdt = w1p.dtype

    base = (Wp + 1) * P
    w1e = w1p[:Cin, base:base + P]
    taps = np.array([kh * Wp + kw for kh in range(3) for kw in range(3)])
    w2r = w2f[:, :P].reshape(3, rowp // P, P, P)
    w2small = w2r[:, taps].reshape(3, 3, 3, P, P)
    w3e = w3b[:P, :Cin]

    w2k = jnp.transpose(w2small, (0, 1, 3, 2, 4)).reshape(9 * P, 3 * P)

    sb12 = jnp.stack([s1p[0, base:base + P], b1p[0, base:base + P],
                      s2t[0, :P], b2t[0, :P]], axis=1)
    sb3 = jnp.stack([s3t[0, :Cin], b3t[0, :Cin]], axis=1)

    x3d = x.reshape(N, Cin, S)
    b_blk = 8
    while N % b_blk:
        b_blk //= 2
    grid = (N // b_blk,)

    ops = (w1e, w2k, w3e, sb12, sb3)
    weight_specs = [pl.BlockSpec(a.shape, lambda g, nd=a.ndim: (0,) * nd)
                    for a in ops]
    in_specs = [pl.BlockSpec((b_blk, Cin, S), lambda g: (g, 0, 0))] + weight_specs
    out_specs = pl.BlockSpec((b_blk, Cin, S), lambda g: (g, 0, 0))

    body = functools.partial(_bottleneck_body, d_size=D, h_size=H, w_size=W)
    y3d = pl.pallas_call(
        body,
        out_shape=jax.ShapeDtypeStruct((N, Cin, S), x.dtype),
        grid_spec=pltpu.PrefetchScalarGridSpec(
            num_scalar_prefetch=0,
            grid=grid,
            in_specs=in_specs,
            out_specs=out_specs,
            scratch_shapes=[
                pltpu.VMEM((b_blk, P, S + 256), cdt),
                pltpu.VMEM((b_blk, 9 * P, S), cdt),
            ]),
        compiler_params=pltpu.CompilerParams(
            dimension_semantics=("parallel",),
            vmem_limit_bytes=64 << 20),
    )(x3d, *ops)

    return y3d.reshape(N, Cin, D, H, W)


# speedup vs baseline: 1.0959x; 1.0959x over previous
import functools

import numpy as np
import jax
import jax.numpy as jnp
from jax.experimental import pallas as pl
from jax.experimental.pallas import tpu as pltpu


def _bottleneck_body(x_ref, w1_ref, w2_ref, w3_ref, sb12_ref, sb3_ref,
                     o_ref, hpad_ref, r2_ref, *, d_size, h_size, w_size):
    b, cin, s_size = x_ref.shape
    p = w1_ref.shape[1]
    cdt = r2_ref.dtype
    f32 = jnp.float32
    dn_b = (((1,), (1,)), ((0,), (0,)))

    x = x_ref[...]

    w1b = jnp.broadcast_to(w1_ref[...][None], (b, cin, p))
    h1 = jax.lax.dot_general(w1b, x.astype(cdt), dn_b,
                             preferred_element_type=f32)
    h1 = jnp.maximum(h1 * sb12_ref[:, 0:1][None] + sb12_ref[:, 1:2][None], 0.0)

    hpad_ref[:, :, 0:128] = jnp.zeros((b, p, 128), cdt)
    hpad_ref[:, :, s_size + 128:s_size + 256] = jnp.zeros((b, p, 128), cdt)
    hpad_ref[:, :, 128:s_size + 128] = h1.astype(cdt)

    lane = jax.lax.broadcasted_iota(jnp.int32, (1, 1, s_size), 2)
    h_of_lane = (lane // w_size) % h_size
    w_of_lane = lane % w_size

    for kd in range(3):
        for kh in range(3):
            t = kd * 3 + kh
            off = 128 + (kd - 1) * h_size * w_size + (kh - 1) * w_size
            src = hpad_ref[:, :, off:off + s_size]
            if kh == 0:
                src = jnp.where(h_of_lane != 0, src, 0)
            elif kh == 2:
                src = jnp.where(h_of_lane != h_size - 1, src, 0)
            r2_ref[:, t * p:(t + 1) * p, :] = src

    r2 = r2_ref[...]
    kp = 9 * p
    w2b = jnp.broadcast_to(w2_ref[...][None], (b, kp, 3 * p))
    y_all = jax.lax.dot_general(w2b, r2, dn_b,
                                preferred_element_type=f32)
    y0 = y_all[:, 0:p]
    y1 = y_all[:, p:2 * p]
    y2 = y_all[:, 2 * p:3 * p]

    h2 = y1
    h2 = h2 + jnp.where(w_of_lane != 0, jnp.roll(y0, 1, axis=2), 0.0)
    h2 = h2 + jnp.where(w_of_lane != w_size - 1, jnp.roll(y2, -1, axis=2), 0.0)
    h2 = jnp.maximum(h2 * sb12_ref[:, 2:3][None] + sb12_ref[:, 3:4][None], 0.0)

    w3b = jnp.broadcast_to(w3_ref[...][None], (b, p, cin))
    h3 = jax.lax.dot_general(w3b, h2.astype(cdt), dn_b,
                             preferred_element_type=f32)
    h3 = h3 * sb3_ref[:, 0:1][None] + sb3_ref[:, 1:2][None]
    o_ref[...] = jnp.maximum(h3 + x, 0.0).astype(o_ref.dtype)


def kernel(x, w1p, s1p, b1p, w2f, s2t, b2t, w3b, s3t, b3t):
    N, Cin, D, H, W = x.shape
    S = D * H * W
    P = w2f.shape[1] // (H * W)
    Wp = W + 2
    rowp = w1p.shape[1]
    cdt = w1p.dtype

    base = (Wp + 1) * P
    w1e = w1p[:Cin, base:base + P]
    taps = np.array([kh * Wp + kw for kh in range(3) for kw in range(3)])
    w2r = w2f[:, :P].reshape(3, rowp // P, P, P)
    w2small = w2r[:, taps].reshape(3, 3, 3, P, P)
    w3e = w3b[:P, :Cin]

    w2k = jnp.transpose(w2small, (0, 1, 3, 2, 4)).reshape(9 * P, 3 * P)

    sb12 = jnp.stack([s1p[0, base:base + P], b1p[0, base:base + P],
                      s2t[0, :P], b2t[0, :P]], axis=1)
    sb3 = jnp.stack([s3t[0, :Cin], b3t[0, :Cin]], axis=1)

    x3d = x.reshape(N, Cin, S)
    b_blk = 16
    while N % b_blk:
        b_blk //= 2
    grid = (N // b_blk,)

    ops = (w1e, w2k, w3e, sb12, sb3)
    weight_specs = [pl.BlockSpec(a.shape, lambda g, nd=a.ndim: (0,) * nd)
                    for a in ops]
    in_specs = [pl.BlockSpec((b_blk, Cin, S), lambda g: (g, 0, 0))] + weight_specs
    out_specs = pl.BlockSpec((b_blk, Cin, S), lambda g: (g, 0, 0))

    body = functools.partial(_bottleneck_body, d_size=D, h_size=H, w_size=W)
    y3d = pl.pallas_call(
        body,
        out_shape=jax.ShapeDtypeStruct((N, Cin, S), x.dtype),
        grid_spec=pltpu.PrefetchScalarGridSpec(
            num_scalar_prefetch=0,
            grid=grid,
            in_specs=in_specs,
            out_specs=out_specs,
            scratch_shapes=[
                pltpu.VMEM((b_blk, P, S + 256), cdt),
                pltpu.VMEM((b_blk, 9 * P, S), cdt),
            ]),
        compiler_params=pltpu.CompilerParams(
            dimension_semantics=("parallel",),
            vmem_limit_bytes=64 << 20),
    )(x3d, *ops)

    return y3d.reshape(N, Cin, D, H, W)
